# Initial kernel scaffold; baseline (speedup 1.0000x reference)
import jax
import jax.numpy as jnp
from jax.experimental import pallas as pl


def kernel(key_len, query_len, weight):
    raise NotImplementedError("write your pallas kernel here")


# speedup vs baseline: 41.7377x; 41.7377x over previous
import math

import jax
import jax.numpy as jnp
from jax import lax
from jax.experimental import pallas as pl
from jax.experimental.pallas import tpu as pltpu
from jax.experimental.pallas import tpu_sc as plsc

H = 16
NB = 32
MAXD = 128
K = 2048
Q = 2048
W = 4224

NSHIFT = 8
GROUPS = 4
KPER = K // NSHIFT // GROUPS


def _table_kernel(w_ref, u8_ref):
    r = pl.program_id(0)
    i = lax.broadcasted_iota(jnp.int32, (H, W), 1)
    n = i - (2040 + r)
    half = NB // 2
    ret = jnp.where(n < 0, half, 0)
    na = jnp.abs(n)
    max_exact = half // 2
    is_small = na < max_exact
    n_safe = jnp.maximum(na, 1)
    val = max_exact + (
        jnp.log(n_safe.astype(jnp.float32) / max_exact)
        / math.log(MAXD / max_exact)
        * (half - max_exact)
    ).astype(jnp.int32)
    val = jnp.minimum(val, half - 1)
    bucket = ret + jnp.where(is_small, na, val)
    acc = jnp.zeros((H, W), jnp.float32)
    for b in range(NB):
        acc = acc + jnp.where(bucket == b, w_ref[:, b : b + 1], 0.0)
    u8_ref[0] = acc


def _sc_body(u8_hbm, out_hbm, u_vmem, sem_in, sem_out):
    c = lax.axis_index("c")
    s = lax.axis_index("s")
    wid = s * 2 + c
    r = wid % NSHIFT
    g = wid // NSHIFT

    pltpu.async_copy(u8_hbm.at[r], u_vmem, sem_in).wait()

    def h_body(h, carry):
        def j_body(j, carry2):
            m = g * KPER + j
            k = r + NSHIFT * m
            base = 2040 - NSHIFT * m
            pltpu.async_copy(
                u_vmem.at[h, pl.ds(base, Q)], out_hbm.at[h, k], sem_out
            )
            return carry2
        return lax.fori_loop(0, KPER, j_body, carry)

    lax.fori_loop(0, H, h_body, 0)

    def drain(idx, carry):
        pltpu.make_async_copy(
            out_hbm.at[0, 0], u_vmem.at[0, pl.ds(0, Q)], sem_out
        ).wait()
        return carry

    lax.fori_loop(0, H * KPER, drain, 0)


def kernel(key_len, query_len, weight):
    del key_len, query_len
    u8 = pl.pallas_call(
        _table_kernel,
        grid=(NSHIFT,),
        in_specs=[pl.BlockSpec((H, NB), lambda r: (0, 0))],
        out_specs=pl.BlockSpec((1, H, W), lambda r: (r, 0, 0)),
        out_shape=jax.ShapeDtypeStruct((NSHIFT, H, W), jnp.float32),
    )(weight)

    mesh = plsc.VectorSubcoreMesh(core_axis_name="c", subcore_axis_name="s")
    out = pl.kernel(
        _sc_body,
        out_type=jax.ShapeDtypeStruct((H, K, Q), jnp.float32),
        mesh=mesh,
        scratch_types=[
            pltpu.VMEM((H, W), jnp.float32),
            pltpu.SemaphoreType.DMA,
            pltpu.SemaphoreType.DMA,
        ],
        compiler_params=pltpu.CompilerParams(use_tc_tiling_on_sc=False),
    )(u8)
    return out


# speedup vs baseline: 42.0588x; 1.0077x over previous
import math

import jax
import jax.numpy as jnp
from jax import lax
from jax.experimental import pallas as pl
from jax.experimental.pallas import tpu as pltpu
from jax.experimental.pallas import tpu_sc as plsc

H = 16
NB = 32
MAXD = 128
K = 2048
Q = 2048
W = 4224

NSHIFT = 8
GROUPS = 4
KPER = K // NSHIFT // GROUPS


def _table_kernel(w_ref, u8_ref):
    r = pl.program_id(0)
    i = lax.broadcasted_iota(jnp.int32, (H, W), 1)
    n = i - (2040 + r)
    half = NB // 2
    ret = jnp.where(n < 0, half, 0)
    na = jnp.abs(n)
    max_exact = half // 2
    is_small = na < max_exact
    n_safe = jnp.maximum(na, 1)
    val = max_exact + (
        jnp.log(n_safe.astype(jnp.float32) / max_exact)
        / math.log(MAXD / max_exact)
        * (half - max_exact)
    ).astype(jnp.int32)
    val = jnp.minimum(val, half - 1)
    bucket = ret + jnp.where(is_small, na, val)
    acc = jnp.zeros((H, W), jnp.float32)
    for b in range(NB):
        acc = acc + jnp.where(bucket == b, w_ref[:, b : b + 1], 0.0)
    u8_ref[0] = acc


def _sc_body(u8_hbm, out_hbm, u_vmem, sem_in, sem_out):
    c = lax.axis_index("c")
    s = lax.axis_index("s")
    wid = s * 2 + c
    r = wid % NSHIFT
    g = wid // NSHIFT

    pltpu.async_copy(u8_hbm.at[r], u_vmem, sem_in).wait()

    def j_body(j, carry):
        m = g * KPER + j
        k = r + NSHIFT * m
        base = 2040 - NSHIFT * m
        pltpu.async_copy(
            u_vmem.at[:, pl.ds(base, Q)], out_hbm.at[:, k], sem_out
        )
        return carry

    lax.fori_loop(0, KPER, j_body, 0)

    def drain(idx, carry):
        pltpu.make_async_copy(
            out_hbm.at[:, 0], u_vmem.at[:, pl.ds(0, Q)], sem_out
        ).wait()
        return carry

    lax.fori_loop(0, KPER, drain, 0)


def kernel(key_len, query_len, weight):
    del key_len, query_len
    u8 = pl.pallas_call(
        _table_kernel,
        grid=(NSHIFT,),
        in_specs=[pl.BlockSpec((H, NB), lambda r: (0, 0))],
        out_specs=pl.BlockSpec((1, H, W), lambda r: (r, 0, 0)),
        out_shape=jax.ShapeDtypeStruct((NSHIFT, H, W), jnp.float32),
    )(weight)

    mesh = plsc.VectorSubcoreMesh(core_axis_name="c", subcore_axis_name="s")
    out = pl.kernel(
        _sc_body,
        out_type=jax.ShapeDtypeStruct((H, K, Q), jnp.float32),
        mesh=mesh,
        scratch_types=[
            pltpu.VMEM((H, W), jnp.float32),
            pltpu.SemaphoreType.DMA,
            pltpu.SemaphoreType.DMA,
        ],
        compiler_params=pltpu.CompilerParams(use_tc_tiling_on_sc=False),
    )(u8)
    return out


# speedup vs baseline: 130.8858x; 3.1120x over previous
import math

import jax
import jax.numpy as jnp
from jax import lax
from jax.experimental import pallas as pl
from jax.experimental.pallas import tpu as pltpu
from jax.experimental.pallas import tpu_sc as plsc

H = 16
NB = 32
MAXD = 128
K = 2048
Q = 2048
W = 4224

NSHIFT = 8
GROUPS = 4
KPER = K // NSHIFT // GROUPS


def _table_kernel(w_ref, u8_ref):
    r = pl.program_id(0)
    i = lax.broadcasted_iota(jnp.int32, (H, W), 1)
    n = i - (2040 + r)
    half = NB // 2
    ret = jnp.where(n < 0, half, 0)
    na = jnp.abs(n)
    max_exact = half // 2
    is_small = na < max_exact
    n_safe = jnp.maximum(na, 1)
    val = max_exact + (
        jnp.log(n_safe.astype(jnp.float32) / max_exact)
        / math.log(MAXD / max_exact)
        * (half - max_exact)
    ).astype(jnp.int32)
    val = jnp.minimum(val, half - 1)
    bucket = ret + jnp.where(is_small, na, val)
    acc = jnp.zeros((H, W), jnp.float32)
    for b in range(NB):
        acc = acc + jnp.where(bucket == b, w_ref[:, b : b + 1], 0.0)
    u8_ref[0] = acc


KT = K // 8
QT = Q // 128
KT_PER = KT // 2


def _sc_body(u8_hbm, out_hbm, u_vmem, sem_in, sem_out):
    c = lax.axis_index("c")
    s = lax.axis_index("s")
    wid = s * 2 + c
    h = wid // 2
    kt0 = (wid % 2) * KT_PER

    pltpu.async_copy(u8_hbm.at[:, h], u_vmem, sem_in).wait()

    def kt_body(j, carry):
        kt = kt0 + j

        def qt_body(qt, carry2):
            i0 = 128 * qt - 8 * kt + 2040
            pltpu.async_copy(
                u_vmem.at[:, pl.ds(i0, 128)], out_hbm.at[h, kt, qt], sem_out
            )
            return carry2

        return lax.fori_loop(0, QT, qt_body, carry)

    lax.fori_loop(0, KT_PER, kt_body, 0)

    def drain(idx, carry):
        pltpu.make_async_copy(
            out_hbm.at[0, 0, 0], u_vmem.at[:, pl.ds(0, 128)], sem_out
        ).wait()
        return carry

    lax.fori_loop(0, KT_PER * QT, drain, 0)


def kernel(key_len, query_len, weight):
    del key_len, query_len
    u8 = pl.pallas_call(
        _table_kernel,
        grid=(NSHIFT,),
        in_specs=[pl.BlockSpec((H, NB), lambda r: (0, 0))],
        out_specs=pl.BlockSpec((1, H, W), lambda r: (r, 0, 0)),
        out_shape=jax.ShapeDtypeStruct((NSHIFT, H, W), jnp.float32),
    )(weight)

    mesh = plsc.VectorSubcoreMesh(core_axis_name="c", subcore_axis_name="s")
    out5 = pl.kernel(
        _sc_body,
        out_type=jax.ShapeDtypeStruct((H, KT, QT, 8, 128), jnp.float32),
        mesh=mesh,
        scratch_types=[
            pltpu.VMEM((NSHIFT, W), jnp.float32),
            pltpu.SemaphoreType.DMA,
            pltpu.SemaphoreType.DMA,
        ],
        compiler_params=pltpu.CompilerParams(use_tc_tiling_on_sc=False),
    )(u8)
    return jnp.transpose(out5, (0, 1, 3, 2, 4)).reshape(H, K, Q)


# speedup vs baseline: 136.2244x; 1.0408x over previous
import math

import jax
import jax.numpy as jnp
from jax import lax
from jax.experimental import pallas as pl
from jax.experimental.pallas import tpu as pltpu
from jax.experimental.pallas import tpu_sc as plsc

H = 16
NB = 32
MAXD = 128
K = 2048
Q = 2048
W = 4224

NSHIFT = 8
GROUPS = 4
KPER = K // NSHIFT // GROUPS


SLAB0 = 1920
SLABW = 384


def _table_kernel(w_ref, u8_ref):
    r = pl.program_id(0)
    i_full = lax.broadcasted_iota(jnp.int32, (H, W), 1)
    u8_ref[0] = jnp.where(
        i_full < 2040 + r, w_ref[:, 31:32], w_ref[:, 15:16]
    )

    i = lax.broadcasted_iota(jnp.int32, (H, SLABW), 1) + SLAB0
    n = i - (2040 + r)
    half = NB // 2
    ret = jnp.where(n < 0, half, 0)
    na = jnp.abs(n)
    max_exact = half // 2
    is_small = na < max_exact
    n_safe = jnp.maximum(na, 1)
    val = max_exact + (
        jnp.log(n_safe.astype(jnp.float32) / max_exact)
        / math.log(MAXD / max_exact)
        * (half - max_exact)
    ).astype(jnp.int32)
    val = jnp.minimum(val, half - 1)
    bucket = ret + jnp.where(is_small, na, val)
    acc = jnp.zeros((H, SLABW), jnp.float32)
    for b in range(NB):
        acc = acc + jnp.where(bucket == b, w_ref[:, b : b + 1], 0.0)
    u8_ref[0, :, SLAB0 : SLAB0 + SLABW] = acc


KT = K // 8
QT = Q // 128
KT_PER = KT // 2


def _sc_body(u8_hbm, out_hbm, u_vmem, sem_in, sem_out):
    c = lax.axis_index("c")
    s = lax.axis_index("s")
    wid = s * 2 + c
    h = wid // 2
    kt0 = (wid % 2) * KT_PER

    pltpu.async_copy(u8_hbm.at[:, h], u_vmem, sem_in).wait()

    def kt_body(j, carry):
        kt = kt0 + j

        def qt_body(qt, carry2):
            i0 = 128 * qt - 8 * kt + 2040
            pltpu.async_copy(
                u_vmem.at[:, pl.ds(i0, 128)], out_hbm.at[h, kt, qt], sem_out
            )
            return carry2

        return lax.fori_loop(0, QT, qt_body, carry)

    lax.fori_loop(0, KT_PER, kt_body, 0)

    def drain(idx, carry):
        pltpu.make_async_copy(
            out_hbm.at[0, 0, 0], u_vmem.at[:, pl.ds(0, 128)], sem_out
        ).wait()
        return carry

    lax.fori_loop(0, KT_PER * QT, drain, 0)


def kernel(key_len, query_len, weight):
    del key_len, query_len
    u8 = pl.pallas_call(
        _table_kernel,
        grid=(NSHIFT,),
        in_specs=[pl.BlockSpec((H, NB), lambda r: (0, 0))],
        out_specs=pl.BlockSpec((1, H, W), lambda r: (r, 0, 0)),
        out_shape=jax.ShapeDtypeStruct((NSHIFT, H, W), jnp.float32),
    )(weight)

    mesh = plsc.VectorSubcoreMesh(core_axis_name="c", subcore_axis_name="s")
    out5 = pl.kernel(
        _sc_body,
        out_type=jax.ShapeDtypeStruct((H, KT, QT, 8, 128), jnp.float32),
        mesh=mesh,
        scratch_types=[
            pltpu.VMEM((NSHIFT, W), jnp.float32),
            pltpu.SemaphoreType.DMA,
            pltpu.SemaphoreType.DMA,
        ],
        compiler_params=pltpu.CompilerParams(use_tc_tiling_on_sc=False),
    )(u8)
    return jnp.transpose(out5, (0, 1, 3, 2, 4)).reshape(H, K, Q)


# speedup vs baseline: 138.9802x; 1.0202x over previous
import math

import jax
import jax.numpy as jnp
from jax import lax
from jax.experimental import pallas as pl
from jax.experimental.pallas import tpu as pltpu
from jax.experimental.pallas import tpu_sc as plsc

H = 16
NB = 32
MAXD = 128
K = 2048
Q = 2048
W = 4480
WL = 3072
NP = 4

NSHIFT = 8
GROUPS = 4
KPER = K // NSHIFT // GROUPS


SLAB0 = 1920
SLABW = 384


def _table_kernel(w_ref, u8_ref):
    r = pl.program_id(0)
    i_full = lax.broadcasted_iota(jnp.int32, (H, W), 1)
    u8_ref[0] = jnp.where(
        i_full < 2040 + r, w_ref[:, 31:32], w_ref[:, 15:16]
    )

    i = lax.broadcasted_iota(jnp.int32, (H, SLABW), 1) + SLAB0
    n = i - (2040 + r)
    half = NB // 2
    ret = jnp.where(n < 0, half, 0)
    na = jnp.abs(n)
    max_exact = half // 2
    is_small = na < max_exact
    n_safe = jnp.maximum(na, 1)
    val = max_exact + (
        jnp.log(n_safe.astype(jnp.float32) / max_exact)
        / math.log(MAXD / max_exact)
        * (half - max_exact)
    ).astype(jnp.int32)
    val = jnp.minimum(val, half - 1)
    bucket = ret + jnp.where(is_small, na, val)
    acc = jnp.zeros((H, SLABW), jnp.float32)
    for b in range(NB):
        acc = acc + jnp.where(bucket == b, w_ref[:, b : b + 1], 0.0)
    u8_ref[0, :, SLAB0 : SLAB0 + SLABW] = acc


KT = K // 8
QT = Q // 128
KT_PER = KT // 2


def _sc_body(u8_hbm, out_hbm, tbl, sem_in, sem_out):
    c = lax.axis_index("c")
    s = lax.axis_index("s")
    wid = s * 2 + c
    h = wid // 2
    kt0 = (wid % 2) * KT_PER
    wl = 1024 - 8 * kt0

    loads = [
        pltpu.async_copy(
            u8_hbm.at[:, h, pl.ds(wl + 128 * p, WL)], tbl.at[p], sem_in
        )
        for p in range(NP)
    ]
    for d in loads:
        d.wait()

    def kt_body(j, carry):
        kt = kt0 + j
        i0 = 1016 - 8 * j

        def q_body(q4, carry2):
            pltpu.async_copy(
                tbl.at[:, :, pl.ds(i0 + 512 * q4, 128)],
                out_hbm.at[h, kt, pl.ds(NP * q4, NP)],
                sem_out,
            )
            return carry2

        return lax.fori_loop(0, QT // NP, q_body, carry)

    lax.fori_loop(0, KT_PER, kt_body, 0)

    def drain(idx, carry):
        pltpu.make_async_copy(
            out_hbm.at[0, 0, pl.ds(0, NP)],
            tbl.at[:, :, pl.ds(0, 128)],
            sem_out,
        ).wait()
        return carry

    lax.fori_loop(0, KT_PER * (QT // NP), drain, 0)


def kernel(key_len, query_len, weight):
    del key_len, query_len
    u8 = pl.pallas_call(
        _table_kernel,
        grid=(NSHIFT,),
        in_specs=[pl.BlockSpec((H, NB), lambda r: (0, 0))],
        out_specs=pl.BlockSpec((1, H, W), lambda r: (r, 0, 0)),
        out_shape=jax.ShapeDtypeStruct((NSHIFT, H, W), jnp.float32),
    )(weight)

    mesh = plsc.VectorSubcoreMesh(core_axis_name="c", subcore_axis_name="s")
    out5 = pl.kernel(
        _sc_body,
        out_type=jax.ShapeDtypeStruct((H, KT, QT, 8, 128), jnp.float32),
        mesh=mesh,
        scratch_types=[
            pltpu.VMEM((NP, NSHIFT, WL), jnp.float32),
            pltpu.SemaphoreType.DMA,
            pltpu.SemaphoreType.DMA,
        ],
        compiler_params=pltpu.CompilerParams(use_tc_tiling_on_sc=False),
    )(u8)
    return jnp.transpose(out5, (0, 1, 3, 2, 4)).reshape(H, K, Q)
